# baseline (speedup 1.0000x reference)
import functools

import numpy as np
import jax
import jax.numpy as jnp
from jax import lax
from jax.experimental import pallas as pl
from jax.experimental.pallas import tpu as pltpu
from jax.experimental.pallas import tpu_sc as plsc

_B = 4096
_D = 32
_S = 25
_NC = 2
_NS = 16
_NW = _NC * _NS
_L = 16
_BPW = _B // _NW
_OPW = _BPW * _S
_NV = _OPW // _L

_PERM = np.asarray(
    [31, 7, 4, 29, 16, 19, 2, 5, 30, 3, 22, 6, 18, 10, 11, 15,
     20, 8, 24, 9, 25, 13, 14, 17, 23, 0, 21, 26, 1, 28, 27, 12],
    dtype=np.int32,
)

_B_IDX = np.repeat(np.arange(_BPW, dtype=np.int32), _S)
_R_IDX = np.tile(np.arange(_S, dtype=np.int32), _BPW)

_mesh = plsc.VectorSubcoreMesh(core_axis_name="c", subcore_axis_name="s")


@functools.partial(
    pl.kernel,
    out_type=jax.ShapeDtypeStruct((_B * _S,), jnp.int32),
    mesh=_mesh,
    compiler_params=pltpu.CompilerParams(
        needs_layout_passes=False, use_tc_tiling_on_sc=False
    ),
    scratch_types=[
        pltpu.VMEM((_BPW,), jnp.int32),
        pltpu.VMEM((_BPW, _D), jnp.int32),
        pltpu.VMEM((_OPW,), jnp.int32),
        pltpu.VMEM((_OPW,), jnp.int32),
        pltpu.VMEM((_OPW,), jnp.int32),
        pltpu.SemaphoreType.DMA,
    ],
)
def _sample_sc(table_hbm, ids_hbm, bi_hbm, ci_hbm, out_hbm,
               ids_v, rows_v, out_v, bi_v, ci_v, sem):
    wid = lax.axis_index("s") * _NC + lax.axis_index("c")
    base = wid * _BPW
    pltpu.sync_copy(bi_hbm, bi_v)
    pltpu.sync_copy(ci_hbm, ci_v)
    pltpu.sync_copy(ids_hbm.at[pl.ds(base, _BPW)], ids_v)
    pltpu.async_copy(table_hbm.at[ids_v], rows_v, sem).wait()

    def body(v, carry):
        bvec = bi_v[pl.ds(v * _L, _L)]
        cvec = ci_v[pl.ds(v * _L, _L)]
        out_v[pl.ds(v * _L, _L)] = plsc.load_gather(rows_v, [bvec, cvec])
        return carry

    lax.fori_loop(0, _NV, body, 0)
    pltpu.sync_copy(out_v, out_hbm.at[pl.ds(base * _S, _OPW)])


def kernel(ids, num_samples, t, adj_info):
    T, N, D = adj_info.shape
    table = adj_info.reshape(T * N, D)
    row_ids = (ids + t * N).astype(jnp.int32)
    cols = lax.dynamic_slice(jnp.asarray(_PERM), (num_samples - _S,), (_S,))
    ci = jnp.take(cols, jnp.asarray(_R_IDX))
    bi = jnp.asarray(_B_IDX)
    out = _sample_sc(table, row_ids, bi, ci)
    return out.reshape(_B, _S)
